# Initial kernel scaffold; baseline (speedup 1.0000x reference)
import jax
import jax.numpy as jnp
from jax.experimental import pallas as pl


def kernel(x, edge_index, W1, b1, W2, b2, W3, b3, gamma, beta, lW1, lb1, lW2, lb2):
    raise NotImplementedError("write your pallas kernel here")


# speedup vs baseline: 34.5431x; 34.5431x over previous
import functools

import jax
import jax.numpy as jnp
from jax import lax
from jax.experimental import pallas as pl
from jax.experimental.pallas import tpu as pltpu
from jax.experimental.pallas import tpu_sc as plsc

N = 10000
E = 320000
F_IN = 128
H = 20
HP = 32
C = 10

NC = 2
NS = 16
NW = NC * NS
B = 128
K = 80
EPT = K * B
E_PAD = NW * EPT
PAD_ROWS = 128
NP = 10240
SP = NP // NS

_mesh = plsc.VectorSubcoreMesh(
    core_axis_name="c", subcore_axis_name="s", num_cores=NC, num_subcores=NS)
_sc_params = pltpu.CompilerParams(use_tc_tiling_on_sc=False)


@functools.partial(
    pl.kernel,
    out_type=jax.ShapeDtypeStruct((NC, NP, 16), jnp.float32),
    mesh=_mesh,
    compiler_params=_sc_params,
    scratch_types=[
        pltpu.VMEM((K, B), jnp.int32),
        pltpu.VMEM((B, 16), jnp.float32),
        pltpu.VMEM_SHARED((NP, 16), jnp.float32),
    ],
)
def _deg_pass(dst_hbm, zeros_hbm, ones_hbm, out_hbm, dst_v, ones_v, acc_sh):
    c = lax.axis_index("c")
    s = lax.axis_index("s")
    wid = s * NC + c
    pltpu.sync_copy(zeros_hbm, acc_sh.at[pl.ds(s * SP, SP)])
    pltpu.sync_copy(ones_hbm, ones_v)
    pltpu.sync_copy(dst_hbm.at[wid], dst_v)
    plsc.subcore_barrier()

    def body(j, carry):
        pltpu.sync_copy(ones_v, acc_sh.at[dst_v.at[j]], add=True)
        return carry

    lax.fori_loop(0, K, body, 0)
    plsc.subcore_barrier()
    pltpu.sync_copy(acc_sh.at[pl.ds(s * SP, SP)],
                    out_hbm.at[c, pl.ds(s * SP, SP)])


@functools.partial(
    pl.kernel,
    out_type=jax.ShapeDtypeStruct((NC, NP, HP), jnp.float32),
    mesh=_mesh,
    compiler_params=_sc_params,
    scratch_types=[
        pltpu.VMEM((K, B), jnp.int32),
        pltpu.VMEM((K, B), jnp.int32),
        pltpu.VMEM((B, HP), jnp.float32),
        pltpu.VMEM((B, HP), jnp.float32),
        pltpu.VMEM_SHARED((NP, HP), jnp.float32),
        pltpu.SemaphoreType.DMA,
        pltpu.SemaphoreType.DMA,
    ],
)
def _agg_pass(u_hbm, src_hbm, dst_hbm, zeros_hbm, out_hbm,
              src_v, dst_v, rows_a, rows_b, acc_sh, sem_a, sem_b):
    c = lax.axis_index("c")
    s = lax.axis_index("s")
    wid = s * NC + c
    pltpu.sync_copy(zeros_hbm, acc_sh.at[pl.ds(s * SP, SP)])
    pltpu.sync_copy(src_hbm.at[wid], src_v)
    pltpu.sync_copy(dst_hbm.at[wid], dst_v)
    plsc.subcore_barrier()

    pltpu.async_copy(u_hbm.at[src_v.at[0]], rows_a, sem_a)

    def body(t, carry):
        ja = 2 * t
        pltpu.make_async_copy(u_hbm.at[src_v.at[ja]], rows_a, sem_a).wait()
        pltpu.async_copy(u_hbm.at[src_v.at[ja + 1]], rows_b, sem_b)
        pltpu.sync_copy(rows_a, acc_sh.at[dst_v.at[ja]], add=True)
        pltpu.make_async_copy(u_hbm.at[src_v.at[ja + 1]], rows_b, sem_b).wait()
        pltpu.async_copy(u_hbm.at[src_v.at[ja + 2]], rows_a, sem_a)
        pltpu.sync_copy(rows_b, acc_sh.at[dst_v.at[ja + 1]], add=True)
        return carry

    lax.fori_loop(0, K // 2 - 1, body, 0)
    pltpu.make_async_copy(u_hbm.at[src_v.at[K - 2]], rows_a, sem_a).wait()
    pltpu.async_copy(u_hbm.at[src_v.at[K - 1]], rows_b, sem_b)
    pltpu.sync_copy(rows_a, acc_sh.at[dst_v.at[K - 2]], add=True)
    pltpu.make_async_copy(u_hbm.at[src_v.at[K - 1]], rows_b, sem_b).wait()
    pltpu.sync_copy(rows_b, acc_sh.at[dst_v.at[K - 1]], add=True)

    plsc.subcore_barrier()
    pltpu.sync_copy(acc_sh.at[pl.ds(s * SP, SP)],
                    out_hbm.at[c, pl.ds(s * SP, SP)])


def _prep_body(x_ref, w_ref, degp_ref, u_ref, dinv_ref):
    z = jnp.dot(x_ref[...], w_ref[...], preferred_element_type=jnp.float32)
    deg = degp_ref[0] + degp_ref[1] + 1.0
    dinv16 = lax.rsqrt(deg)
    dinv = jnp.concatenate([dinv16, dinv16], axis=1)
    zp = jnp.concatenate(
        [z, jnp.zeros((NP - N, HP), jnp.float32)], axis=0)
    u_ref[...] = zp * dinv
    dinv_ref[...] = dinv


def _layer_tail(accp_ref, u_ref, dinv_ref, b_ref, g_ref, be_ref):
    y = dinv_ref[...] * (accp_ref[0] + accp_ref[1] + u_ref[...]) + b_ref[...]
    mask = lax.broadcasted_iota(jnp.int32, (NP, HP), 0) < N
    y = jnp.where(mask, y, 0.0)
    mu = jnp.sum(y, axis=0, keepdims=True) * (1.0 / N)
    var = jnp.sum(y * y, axis=0, keepdims=True) * (1.0 / N) - mu * mu
    h = (y - mu) * lax.rsqrt(var + 1e-5) * g_ref[...] + be_ref[...]
    return jnp.where(mask, jnp.maximum(h, 0.0), 0.0)


def _mid_body(accp_ref, u_ref, dinv_ref, b_ref, g_ref, be_ref, w_ref, out_ref):
    h = _layer_tail(accp_ref, u_ref, dinv_ref, b_ref, g_ref, be_ref)
    out_ref[...] = dinv_ref[...] * jnp.dot(
        h, w_ref[...], preferred_element_type=jnp.float32)


def _fin_body(accp_ref, u_ref, dinv_ref, b_ref, g_ref, be_ref,
              lw1_ref, lb1_ref, lw2_ref, lb2_ref, out_ref):
    h = _layer_tail(accp_ref, u_ref, dinv_ref, b_ref, g_ref, be_ref)
    gmax = jnp.max(h, axis=0, keepdims=True)
    g1 = jnp.maximum(
        jnp.dot(gmax, lw1_ref[...], preferred_element_type=jnp.float32)
        + lb1_ref[...], 0.0)
    out_ref[...] = jnp.dot(
        g1, lw2_ref[...], preferred_element_type=jnp.float32) + lb2_ref[...]


_f32 = jnp.float32
_prep = pl.pallas_call(
    _prep_body,
    out_shape=(jax.ShapeDtypeStruct((NP, HP), _f32),
               jax.ShapeDtypeStruct((NP, HP), _f32)))
_mid = pl.pallas_call(
    _mid_body, out_shape=jax.ShapeDtypeStruct((NP, HP), _f32))
_fin = pl.pallas_call(
    _fin_body, out_shape=jax.ShapeDtypeStruct((1, C), _f32))


def _pad2(a, rows, cols):
    out = jnp.zeros((rows, cols), jnp.float32)
    return lax.dynamic_update_slice(out, a, (0, 0))


def kernel(x, edge_index, W1, b1, W2, b2, W3, b3, gamma, beta,
           lW1, lb1, lW2, lb2):
    src = edge_index[0]
    dst = edge_index[1]
    npad = E_PAD - E
    padi = jnp.arange(npad, dtype=jnp.int32)
    src_p = jnp.concatenate([src, (padi * 131) % N])
    dst_p = jnp.concatenate([dst, N + (padi % PAD_ROWS)])
    src3 = src_p.reshape(NW, K, B)
    dst3 = dst_p.reshape(NW, K, B)

    zeros_sp = jnp.zeros((SP, HP), jnp.float32)
    zeros_sp16 = jnp.zeros((SP, 16), jnp.float32)
    ones_b16 = jnp.ones((B, 16), jnp.float32)

    W1p = _pad2(W1, F_IN, HP)
    W2p = _pad2(W2, HP, HP)
    W3p = _pad2(W3, HP, HP)
    lW1p = _pad2(lW1, HP, HP)
    lW2p = _pad2(lW2, HP, C)
    b1p = _pad2(b1[None, :], 1, HP)
    b2p = _pad2(b2[None, :], 1, HP)
    b3p = _pad2(b3[None, :], 1, HP)
    gp = _pad2(gamma[None, :], 1, HP)
    bep = _pad2(beta[None, :], 1, HP)
    lb1p = _pad2(lb1[None, :], 1, HP)
    lb2p = lb2[None, :]

    degp = _deg_pass(dst3, zeros_sp16, ones_b16)
    u1, dinvb = _prep(x, W1p, degp)
    acc1 = _agg_pass(u1, src3, dst3, zeros_sp)
    u2 = _mid(acc1, u1, dinvb, b1p, gp, bep, W2p)
    acc2 = _agg_pass(u2, src3, dst3, zeros_sp)
    u3 = _mid(acc2, u2, dinvb, b2p, gp, bep, W3p)
    acc3 = _agg_pass(u3, src3, dst3, zeros_sp)
    return _fin(acc3, u3, dinvb, b3p, gp, bep, lW1p, lb1p, lW2p, lb2p)


# speedup vs baseline: 50.2962x; 1.4560x over previous
import functools

import jax
import jax.numpy as jnp
from jax import lax
from jax.experimental import pallas as pl
from jax.experimental.pallas import tpu as pltpu
from jax.experimental.pallas import tpu_sc as plsc

N = 10000
E = 320000
F_IN = 128
H = 20
HP = 32
C = 10

NC = 2
NS = 16
NW = NC * NS
B = 128
K = 80
EPT = K * B
E_PAD = NW * EPT
PAD_ROWS = 128
NP = 10240
SP = NP // NS

_mesh = plsc.VectorSubcoreMesh(
    core_axis_name="c", subcore_axis_name="s", num_cores=NC, num_subcores=NS)
_sc_params = pltpu.CompilerParams(use_tc_tiling_on_sc=False)


@functools.partial(
    pl.kernel,
    out_type=jax.ShapeDtypeStruct((NC, NP, 16), jnp.float32),
    mesh=_mesh,
    compiler_params=_sc_params,
    scratch_types=[
        pltpu.VMEM((K, B), jnp.int32),
        pltpu.VMEM((B, 16), jnp.float32),
        pltpu.VMEM_SHARED((NP, 16), jnp.float32),
    ],
)
def _deg_pass(dst_hbm, zeros_hbm, ones_hbm, out_hbm, dst_v, ones_v, acc_sh):
    c = lax.axis_index("c")
    s = lax.axis_index("s")
    wid = s * NC + c
    pltpu.sync_copy(zeros_hbm, acc_sh.at[pl.ds(s * SP, SP)])
    pltpu.sync_copy(ones_hbm, ones_v)
    pltpu.sync_copy(dst_hbm.at[wid], dst_v)
    plsc.subcore_barrier()

    def body(j, carry):
        pltpu.sync_copy(ones_v, acc_sh.at[dst_v.at[j]], add=True)
        return carry

    lax.fori_loop(0, K, body, 0)
    plsc.subcore_barrier()
    pltpu.sync_copy(acc_sh.at[pl.ds(s * SP, SP)],
                    out_hbm.at[c, pl.ds(s * SP, SP)])


RING = 8
AHEAD = 4


@functools.partial(
    pl.kernel,
    out_type=jax.ShapeDtypeStruct((NC, NP, HP), jnp.float32),
    mesh=_mesh,
    compiler_params=_sc_params,
    scratch_types=[
        pltpu.VMEM((K, B), jnp.int32),
        pltpu.VMEM((K, B), jnp.int32),
        pltpu.VMEM((RING, B, HP), jnp.float32),
        pltpu.VMEM_SHARED((NP, HP), jnp.float32),
        pltpu.SemaphoreType.DMA((RING,)),
        pltpu.SemaphoreType.DMA((RING,)),
    ],
)
def _agg_pass(u_hbm, src_hbm, dst_hbm, zeros_hbm, out_hbm,
              src_v, dst_v, rows, acc_sh, sem_g, sem_s):
    c = lax.axis_index("c")
    s = lax.axis_index("s")
    wid = s * NC + c
    pltpu.sync_copy(zeros_hbm, acc_sh.at[pl.ds(s * SP, SP)])
    pltpu.sync_copy(src_hbm.at[wid], src_v)
    pltpu.sync_copy(dst_hbm.at[wid], dst_v)
    plsc.subcore_barrier()

    def gather(i, b):
        pltpu.async_copy(u_hbm.at[src_v.at[i]], rows.at[b], sem_g.at[b])

    def wait_gather(i, b):
        pltpu.make_async_copy(
            u_hbm.at[src_v.at[i]], rows.at[b], sem_g.at[b]).wait()

    def scatter(i, b):
        pltpu.async_copy(rows.at[b], acc_sh.at[dst_v.at[i]], sem_s.at[b],
                         add=True)

    def wait_scatter(i, b):
        pltpu.make_async_copy(
            rows.at[b], acc_sh.at[dst_v.at[i]], sem_s.at[b]).wait()

    for b in range(AHEAD):
        gather(b, b)
    for b in range(RING):
        wait_gather(b, b)
        scatter(b, b)
        if b >= AHEAD:
            wait_scatter(b - AHEAD, b - AHEAD)
        gather(b + AHEAD, (b + AHEAD) % RING)

    def body(t, carry):
        i0 = t * RING
        for b in range(RING):
            i = i0 + b
            wait_gather(i, b)
            scatter(i, b)
            wait_scatter(i - AHEAD, (b - AHEAD) % RING)
            gather(i + AHEAD, (b + AHEAD) % RING)
        return carry

    lax.fori_loop(1, K // RING - 1, body, 0)

    i0 = K - RING
    for b in range(RING):
        i = i0 + b
        wait_gather(i, b)
        scatter(i, b)
        wait_scatter(i - AHEAD, (b - AHEAD) % RING)
        if i + AHEAD < K:
            gather(i + AHEAD, (b + AHEAD) % RING)
    for b in range(RING - AHEAD, RING):
        wait_scatter(i0 + b, b)

    plsc.subcore_barrier()
    pltpu.sync_copy(acc_sh.at[pl.ds(s * SP, SP)],
                    out_hbm.at[c, pl.ds(s * SP, SP)])


def _prep_body(x_ref, w_ref, degp_ref, u_ref, dinv_ref):
    z = jnp.dot(x_ref[...], w_ref[...], preferred_element_type=jnp.float32)
    deg = degp_ref[0] + degp_ref[1] + 1.0
    dinv16 = lax.rsqrt(deg)
    dinv = jnp.concatenate([dinv16, dinv16], axis=1)
    zp = jnp.concatenate(
        [z, jnp.zeros((NP - N, HP), jnp.float32)], axis=0)
    u_ref[...] = zp * dinv
    dinv_ref[...] = dinv


def _layer_tail(accp_ref, u_ref, dinv_ref, b_ref, g_ref, be_ref):
    y = dinv_ref[...] * (accp_ref[0] + accp_ref[1] + u_ref[...]) + b_ref[...]
    mask = lax.broadcasted_iota(jnp.int32, (NP, HP), 0) < N
    y = jnp.where(mask, y, 0.0)
    mu = jnp.sum(y, axis=0, keepdims=True) * (1.0 / N)
    var = jnp.sum(y * y, axis=0, keepdims=True) * (1.0 / N) - mu * mu
    h = (y - mu) * lax.rsqrt(var + 1e-5) * g_ref[...] + be_ref[...]
    return jnp.where(mask, jnp.maximum(h, 0.0), 0.0)


def _mid_body(accp_ref, u_ref, dinv_ref, b_ref, g_ref, be_ref, w_ref, out_ref):
    h = _layer_tail(accp_ref, u_ref, dinv_ref, b_ref, g_ref, be_ref)
    out_ref[...] = dinv_ref[...] * jnp.dot(
        h, w_ref[...], preferred_element_type=jnp.float32)


def _fin_body(accp_ref, u_ref, dinv_ref, b_ref, g_ref, be_ref,
              lw1_ref, lb1_ref, lw2_ref, lb2_ref, out_ref):
    h = _layer_tail(accp_ref, u_ref, dinv_ref, b_ref, g_ref, be_ref)
    gmax = jnp.max(h, axis=0, keepdims=True)
    g1 = jnp.maximum(
        jnp.dot(gmax, lw1_ref[...], preferred_element_type=jnp.float32)
        + lb1_ref[...], 0.0)
    out_ref[...] = jnp.dot(
        g1, lw2_ref[...], preferred_element_type=jnp.float32) + lb2_ref[...]


_f32 = jnp.float32
_prep = pl.pallas_call(
    _prep_body,
    out_shape=(jax.ShapeDtypeStruct((NP, HP), _f32),
               jax.ShapeDtypeStruct((NP, HP), _f32)))
_mid = pl.pallas_call(
    _mid_body, out_shape=jax.ShapeDtypeStruct((NP, HP), _f32))
_fin = pl.pallas_call(
    _fin_body, out_shape=jax.ShapeDtypeStruct((1, C), _f32))


def _pad2(a, rows, cols):
    out = jnp.zeros((rows, cols), jnp.float32)
    return lax.dynamic_update_slice(out, a, (0, 0))


def kernel(x, edge_index, W1, b1, W2, b2, W3, b3, gamma, beta,
           lW1, lb1, lW2, lb2):
    src = edge_index[0]
    dst = edge_index[1]
    npad = E_PAD - E
    padi = jnp.arange(npad, dtype=jnp.int32)
    src_p = jnp.concatenate([src, (padi * 131) % N])
    dst_p = jnp.concatenate([dst, N + (padi % PAD_ROWS)])
    src3 = src_p.reshape(NW, K, B)
    dst3 = dst_p.reshape(NW, K, B)

    zeros_sp = jnp.zeros((SP, HP), jnp.float32)
    zeros_sp16 = jnp.zeros((SP, 16), jnp.float32)
    ones_b16 = jnp.ones((B, 16), jnp.float32)

    W1p = _pad2(W1, F_IN, HP)
    W2p = _pad2(W2, HP, HP)
    W3p = _pad2(W3, HP, HP)
    lW1p = _pad2(lW1, HP, HP)
    lW2p = _pad2(lW2, HP, C)
    b1p = _pad2(b1[None, :], 1, HP)
    b2p = _pad2(b2[None, :], 1, HP)
    b3p = _pad2(b3[None, :], 1, HP)
    gp = _pad2(gamma[None, :], 1, HP)
    bep = _pad2(beta[None, :], 1, HP)
    lb1p = _pad2(lb1[None, :], 1, HP)
    lb2p = lb2[None, :]

    degp = _deg_pass(dst3, zeros_sp16, ones_b16)
    u1, dinvb = _prep(x, W1p, degp)
    acc1 = _agg_pass(u1, src3, dst3, zeros_sp)
    u2 = _mid(acc1, u1, dinvb, b1p, gp, bep, W2p)
    acc2 = _agg_pass(u2, src3, dst3, zeros_sp)
    u3 = _mid(acc2, u2, dinvb, b2p, gp, bep, W3p)
    acc3 = _agg_pass(u3, src3, dst3, zeros_sp)
    return _fin(acc3, u3, dinvb, b3p, gp, bep, lW1p, lb1p, lW2p, lb2p)
